# Initial kernel scaffold; baseline (speedup 1.0000x reference)
import jax
import jax.numpy as jnp
from jax.experimental import pallas as pl


def kernel(rnode_features, pnode_features, edge_features, senders, receivers, We1, be1, We2, be2, Wu1, bu1, Wu2, bu2, Wn1, bn1, Wn2, bn2, Wo1, bo1, Wo2, bo2):
    raise NotImplementedError("write your pallas kernel here")


# speedup vs baseline: 3.1521x; 3.1521x over previous
import functools

import jax
import jax.numpy as jnp
from jax import lax
from jax.experimental import pallas as pl
from jax.experimental.pallas import tpu as pltpu
from jax.experimental.pallas import tpu_sc as plsc

N = 10000
E = 160000
D = 128
H = 128
DE = 4
OUT = 3

NC = 2
NS = 16
NW = NC * NS

W = 128
EPAD = 163840
PER_W = EPAD // NW
PAD = EPAD - E
DUMMY = 624
NSEG = N + DUMMY


def _ln(x):
    m = jnp.mean(x, axis=-1, keepdims=True)
    v = jnp.mean((x - m) * (x - m), axis=-1, keepdims=True)
    return (x - m) * lax.rsqrt(v + 1e-5)


def _swish(x):
    return x * jax.nn.sigmoid(x)


def _proj_body(rn_ref, pn_ref, wb_ref, wc_ref, sp_ref, rp_ref):
    sp_ref[...] = jnp.dot(rn_ref[...], wb_ref[...],
                          preferred_element_type=jnp.float32)
    rp_ref[...] = jnp.dot(pn_ref[...], wc_ref[...],
                          preferred_element_type=jnp.float32)


def _node_proj(rn, pn, wu1b, wu1c):
    tile = 2000
    grid = N // tile
    return pl.pallas_call(
        _proj_body,
        grid=(grid,),
        in_specs=[
            pl.BlockSpec((tile, D), lambda i: (i, 0)),
            pl.BlockSpec((tile, D), lambda i: (i, 0)),
            pl.BlockSpec((D, H), lambda i: (0, 0)),
            pl.BlockSpec((D, H), lambda i: (0, 0)),
        ],
        out_specs=[
            pl.BlockSpec((tile, H), lambda i: (i, 0)),
            pl.BlockSpec((tile, H), lambda i: (i, 0)),
        ],
        out_shape=[
            jax.ShapeDtypeStruct((N, H), jnp.float32),
            jax.ShapeDtypeStruct((N, H), jnp.float32),
        ],
        compiler_params=pltpu.CompilerParams(
            dimension_semantics=("parallel",)),
    )(rn, pn, wu1b, wu1c)


def _gather_body(st_hbm, rt_hbm, si_hbm, ri_hbm, so_hbm, ro_hbm,
                 idx_s, idx_r, rows_s, rows_r, sem_s, sem_r):
    wid = lax.axis_index("s") * NC + lax.axis_index("c")
    base = wid * PER_W

    @pl.loop(0, PER_W, step=W)
    def _(i):
        off = base + i
        pltpu.sync_copy(si_hbm.at[pl.ds(off, W)], idx_s)
        pltpu.sync_copy(ri_hbm.at[pl.ds(off, W)], idx_r)
        cs = pltpu.async_copy(st_hbm.at[idx_s], rows_s, sem_s)
        cr = pltpu.async_copy(rt_hbm.at[idx_r], rows_r, sem_r)
        cs.wait()
        cr.wait()
        pltpu.sync_copy(rows_s, so_hbm.at[pl.ds(off, W)])
        pltpu.sync_copy(rows_r, ro_hbm.at[pl.ds(off, W)])


def _sc_gather(sndp_t, rcvp_t, senders_p, receivers_g):
    mesh = plsc.VectorSubcoreMesh(core_axis_name="c", subcore_axis_name="s")
    f = pl.kernel(
        _gather_body,
        out_type=[
            jax.ShapeDtypeStruct((EPAD, H), jnp.float32),
            jax.ShapeDtypeStruct((EPAD, H), jnp.float32),
        ],
        mesh=mesh,
        scratch_types=[
            pltpu.VMEM((W,), jnp.int32),
            pltpu.VMEM((W,), jnp.int32),
            pltpu.VMEM((W, H), jnp.float32),
            pltpu.VMEM((W, H), jnp.float32),
            pltpu.SemaphoreType.DMA,
            pltpu.SemaphoreType.DMA,
        ],
    )
    return f(sndp_t, rcvp_t, senders_p, receivers_g)


def _count_body(rs_hbm, zcnt_hbm, cnt_out, idx_c, ones_v, cnt_sh):
    cid = lax.axis_index("c")
    sid = lax.axis_index("s")
    wid = sid * NC + cid
    base = wid * PER_W
    rows = NSEG // NS

    @pl.loop(0, W)
    def _(i):
        @pl.loop(0, D, step=16)
        def _(j):
            ones_v[i, pl.ds(j, 16)] = jnp.full((16,), 1.0, jnp.float32)

    pltpu.sync_copy(zcnt_hbm.at[pl.ds(sid * rows, rows)],
                    cnt_sh.at[pl.ds(sid * rows, rows)])
    plsc.subcore_barrier()

    @pl.loop(0, PER_W, step=W)
    def _(i):
        pltpu.sync_copy(rs_hbm.at[pl.ds(base + i, W)], idx_c)
        pltpu.sync_copy(ones_v, cnt_sh.at[idx_c], add=True)

    plsc.subcore_barrier()
    pltpu.sync_copy(cnt_sh.at[pl.ds(sid * rows, rows)],
                    cnt_out.at[cid].at[pl.ds(sid * rows, rows)])


def _sc_count(receivers_s, zcnt):
    mesh = plsc.VectorSubcoreMesh(core_axis_name="c", subcore_axis_name="s")
    f = pl.kernel(
        _count_body,
        out_type=jax.ShapeDtypeStruct((NC, NSEG, D), jnp.float32),
        mesh=mesh,
        scratch_types=[
            pltpu.VMEM((W,), jnp.int32),
            pltpu.VMEM((W, D), jnp.float32),
            pltpu.VMEM_SHARED((NSEG, D), jnp.float32),
        ],
    )
    return f(receivers_s, zcnt)


def _edge_body(ef_ref, sp_ref, rp_ref, we1_ref, be1_ref, we2_ref, be2_ref,
               wu1a_ref, bu1_ref, wu2_ref, bu2_ref, out_ref):
    x = ef_ref[...]
    h1 = jnp.dot(x, we1_ref[...], preferred_element_type=jnp.float32)
    h1 = _swish(h1 + be1_ref[...])
    e0 = jnp.dot(h1, we2_ref[...], preferred_element_type=jnp.float32)
    e0 = _ln(e0 + be2_ref[...])
    pre = jnp.dot(e0, wu1a_ref[...], preferred_element_type=jnp.float32)
    pre = _swish(pre + sp_ref[...] + rp_ref[...] + bu1_ref[...])
    u = jnp.dot(pre, wu2_ref[...], preferred_element_type=jnp.float32)
    out_ref[...] = e0 + _ln(u + bu2_ref[...])


def _edge_mlp(ef_p, sndp, rcvp, we1, be1, we2, be2, wu1a, bu1, wu2, bu2):
    tile = 1024
    grid = EPAD // tile
    full = lambda i: (0, 0)
    return pl.pallas_call(
        _edge_body,
        grid=(grid,),
        in_specs=[
            pl.BlockSpec((tile, DE), lambda i: (i, 0)),
            pl.BlockSpec((tile, H), lambda i: (i, 0)),
            pl.BlockSpec((tile, H), lambda i: (i, 0)),
            pl.BlockSpec((DE, H), full),
            pl.BlockSpec((1, H), full),
            pl.BlockSpec((H, D), full),
            pl.BlockSpec((1, D), full),
            pl.BlockSpec((D, H), full),
            pl.BlockSpec((1, H), full),
            pl.BlockSpec((H, D), full),
            pl.BlockSpec((1, D), full),
        ],
        out_specs=pl.BlockSpec((tile, D), lambda i: (i, 0)),
        out_shape=jax.ShapeDtypeStruct((EPAD, D), jnp.float32),
        compiler_params=pltpu.CompilerParams(
            dimension_semantics=("parallel",)),
    )(ef_p, sndp, rcvp, we1, be1, we2, be2, wu1a, bu1, wu2, bu2)


def _scatter_body(vals_hbm, ridx_hbm, zseg_hbm, seg_out,
                  idx_v, vals_v, sem, seg_sh):
    cid = lax.axis_index("c")
    sid = lax.axis_index("s")
    wid = sid * NC + cid
    base = wid * PER_W
    rows = NSEG // NS

    pltpu.sync_copy(zseg_hbm.at[pl.ds(sid * rows, rows)],
                    seg_sh.at[pl.ds(sid * rows, rows)])
    plsc.subcore_barrier()

    @pl.loop(0, PER_W, step=W)
    def _(i):
        off = base + i
        pltpu.sync_copy(ridx_hbm.at[pl.ds(off, W)], idx_v)
        pltpu.async_copy(vals_hbm.at[pl.ds(off, W)], vals_v, sem).wait()
        pltpu.sync_copy(vals_v, seg_sh.at[idx_v], add=True)

    plsc.subcore_barrier()

    pltpu.sync_copy(seg_sh.at[pl.ds(sid * rows, rows)],
                    seg_out.at[cid].at[pl.ds(sid * rows, rows)])


def _sc_scatter(vals, receivers_s, zseg):
    mesh = plsc.VectorSubcoreMesh(core_axis_name="c", subcore_axis_name="s")
    f = pl.kernel(
        _scatter_body,
        out_type=jax.ShapeDtypeStruct((NC, NSEG, D), jnp.float32),
        mesh=mesh,
        scratch_types=[
            pltpu.VMEM((W,), jnp.int32),
            pltpu.VMEM((W, D), jnp.float32),
            pltpu.SemaphoreType.DMA,
            pltpu.VMEM_SHARED((NSEG, D), jnp.float32),
        ],
    )
    return f(vals, receivers_s, zseg)


def _node_body(pn_ref, seg_ref, cnt_ref, wn1a_ref, wn1b_ref, bn1_ref,
               wn2_ref, bn2_ref, wo1_ref, bo1_ref, wo2_ref, bo2_ref,
               out_ref):
    pn = pn_ref[...]
    seg = seg_ref[0] + seg_ref[1]
    cnt = cnt_ref[0] + cnt_ref[1]
    agg = seg / jnp.maximum(cnt, 1.0)
    pre = (jnp.dot(pn, wn1a_ref[...], preferred_element_type=jnp.float32)
           + jnp.dot(agg, wn1b_ref[...], preferred_element_type=jnp.float32))
    h = _swish(pre + bn1_ref[...])
    u = jnp.dot(h, wn2_ref[...], preferred_element_type=jnp.float32)
    node = pn + _ln(u + bn2_ref[...])
    h2 = _swish(jnp.dot(node, wo1_ref[...],
                        preferred_element_type=jnp.float32) + bo1_ref[...])
    out_ref[...] = jnp.dot(h2, wo2_ref[...],
                           preferred_element_type=jnp.float32) + bo2_ref[...]


def _node_mlp(pn, seg_parts, cnt_parts, wn1a, wn1b, bn1, wn2, bn2,
              wo1, bo1, wo2, bo2):
    tile = 2000
    grid = N // tile
    full = lambda i: (0, 0)
    return pl.pallas_call(
        _node_body,
        grid=(grid,),
        in_specs=[
            pl.BlockSpec((tile, D), lambda i: (i, 0)),
            pl.BlockSpec((NC, tile, D), lambda i: (0, i, 0)),
            pl.BlockSpec((NC, tile, D), lambda i: (0, i, 0)),
            pl.BlockSpec((D, H), full),
            pl.BlockSpec((D, H), full),
            pl.BlockSpec((1, H), full),
            pl.BlockSpec((H, D), full),
            pl.BlockSpec((1, D), full),
            pl.BlockSpec((D, H), full),
            pl.BlockSpec((1, H), full),
            pl.BlockSpec((H, OUT), full),
            pl.BlockSpec((1, OUT), full),
        ],
        out_specs=pl.BlockSpec((tile, OUT), lambda i: (i, 0)),
        out_shape=jax.ShapeDtypeStruct((N, OUT), jnp.float32),
        compiler_params=pltpu.CompilerParams(
            dimension_semantics=("parallel",)),
    )(pn, seg_parts, cnt_parts, wn1a, wn1b, bn1, wn2, bn2,
      wo1, bo1, wo2, bo2)


def kernel(rnode_features, pnode_features, edge_features, senders, receivers,
           We1, be1, We2, be2, Wu1, bu1, Wu2, bu2,
           Wn1, bn1, Wn2, bn2, Wo1, bo1, Wo2, bo2):
    rn = rnode_features.reshape(N, D)
    pn = pnode_features.reshape(N, D)
    ef = edge_features.astype(jnp.float32)

    pad_g = (jnp.arange(PAD, dtype=jnp.int32) * 37) % N
    pad_s = N + (jnp.arange(PAD, dtype=jnp.int32) % DUMMY)
    senders_p = jnp.concatenate([senders, pad_g])
    receivers_g = jnp.concatenate([receivers, pad_g])
    receivers_s = jnp.concatenate([receivers, pad_s])
    ef_p = jnp.zeros((EPAD, DE), jnp.float32).at[:E].set(ef)

    wu1a, wu1b, wu1c = Wu1[:D], Wu1[D:2 * D], Wu1[2 * D:]
    wn1a, wn1b = Wn1[:D], Wn1[D:]
    r2 = lambda b: b.reshape(1, -1)

    zseg = jnp.zeros((NSEG, D), jnp.float32)
    sndp_t, rcvp_t = _node_proj(rn, pn, wu1b, wu1c)
    sndp, rcvp = _sc_gather(sndp_t, rcvp_t, senders_p, receivers_g)
    cnt_parts = _sc_count(receivers_s, zseg)
    e1 = _edge_mlp(ef_p, sndp, rcvp, We1, r2(be1), We2, r2(be2),
                   wu1a, r2(bu1), Wu2, r2(bu2))
    seg_parts = _sc_scatter(e1, receivers_s, zseg)
    out = _node_mlp(pn, seg_parts, cnt_parts, wn1a, wn1b, r2(bn1),
                    Wn2, r2(bn2), Wo1, r2(bo1), Wo2, r2(bo2))
    return out.reshape(N, 1, OUT)


# speedup vs baseline: 3.5183x; 1.1162x over previous
import functools

import jax
import jax.numpy as jnp
from jax import lax
from jax.experimental import pallas as pl
from jax.experimental.pallas import tpu as pltpu
from jax.experimental.pallas import tpu_sc as plsc

N = 10000
E = 160000
D = 128
H = 128
DE = 4
OUT = 3

NC = 2
NS = 16
NW = NC * NS

W = 128
EPAD = 163840
PER_W = EPAD // NW
PAD = EPAD - E
DUMMY = 624
NSEG = N + DUMMY


def _ln(x):
    m = jnp.mean(x, axis=-1, keepdims=True)
    v = jnp.mean((x - m) * (x - m), axis=-1, keepdims=True)
    return (x - m) * lax.rsqrt(v + 1e-5)


def _swish(x):
    return x * jax.nn.sigmoid(x)


def _bdot(a, b):
    return jnp.dot(a.astype(jnp.bfloat16), b.astype(jnp.bfloat16),
                   preferred_element_type=jnp.float32)


def _proj_body(rn_ref, pn_ref, wb_ref, wc_ref, sp_ref, rp_ref):
    sp_ref[...] = _bdot(rn_ref[...], wb_ref[...])
    rp_ref[...] = _bdot(pn_ref[...], wc_ref[...])


def _node_proj(rn, pn, wu1b, wu1c):
    tile = 2000
    grid = N // tile
    return pl.pallas_call(
        _proj_body,
        grid=(grid,),
        in_specs=[
            pl.BlockSpec((tile, D), lambda i: (i, 0)),
            pl.BlockSpec((tile, D), lambda i: (i, 0)),
            pl.BlockSpec((D, H), lambda i: (0, 0)),
            pl.BlockSpec((D, H), lambda i: (0, 0)),
        ],
        out_specs=[
            pl.BlockSpec((tile, H), lambda i: (i, 0)),
            pl.BlockSpec((tile, H), lambda i: (i, 0)),
        ],
        out_shape=[
            jax.ShapeDtypeStruct((N, H), jnp.float32),
            jax.ShapeDtypeStruct((N, H), jnp.float32),
        ],
        compiler_params=pltpu.CompilerParams(
            dimension_semantics=("parallel",)),
    )(rn, pn, wu1b, wu1c)


def _gather_body(st_hbm, rt_hbm, si_hbm, ri_hbm, so_hbm, ro_hbm,
                 idx_s0, idx_r0, rows_s0, rows_r0, sem_s0, sem_r0,
                 idx_s1, idx_r1, rows_s1, rows_r1, sem_s1, sem_r1):
    wid = lax.axis_index("s") * NC + lax.axis_index("c")
    base = wid * PER_W

    def start(off, idx_s, idx_r, rows_s, rows_r, sem_s, sem_r):
        pltpu.sync_copy(si_hbm.at[pl.ds(off, W)], idx_s)
        pltpu.sync_copy(ri_hbm.at[pl.ds(off, W)], idx_r)
        pltpu.async_copy(st_hbm.at[idx_s], rows_s, sem_s)
        pltpu.async_copy(rt_hbm.at[idx_r], rows_r, sem_r)

    def drain(off, idx_s, idx_r, rows_s, rows_r, sem_s, sem_r):
        pltpu.make_async_copy(st_hbm.at[idx_s], rows_s, sem_s).wait()
        pltpu.make_async_copy(rt_hbm.at[idx_r], rows_r, sem_r).wait()
        pltpu.sync_copy(rows_s, so_hbm.at[pl.ds(off, W)])
        pltpu.sync_copy(rows_r, ro_hbm.at[pl.ds(off, W)])

    b0 = (idx_s0, idx_r0, rows_s0, rows_r0, sem_s0, sem_r0)
    b1 = (idx_s1, idx_r1, rows_s1, rows_r1, sem_s1, sem_r1)

    start(base, *b0)

    @pl.loop(0, PER_W - 2 * W, step=2 * W)
    def _(i):
        start(base + i + W, *b1)
        drain(base + i, *b0)
        start(base + i + 2 * W, *b0)
        drain(base + i + W, *b1)

    start(base + PER_W - W, *b1)
    drain(base + PER_W - 2 * W, *b0)
    drain(base + PER_W - W, *b1)


def _sc_gather(sndp_t, rcvp_t, senders_p, receivers_g):
    mesh = plsc.VectorSubcoreMesh(core_axis_name="c", subcore_axis_name="s")
    buf = [
        pltpu.VMEM((W,), jnp.int32),
        pltpu.VMEM((W,), jnp.int32),
        pltpu.VMEM((W, H), jnp.float32),
        pltpu.VMEM((W, H), jnp.float32),
        pltpu.SemaphoreType.DMA,
        pltpu.SemaphoreType.DMA,
    ]
    f = pl.kernel(
        _gather_body,
        out_type=[
            jax.ShapeDtypeStruct((EPAD, H), jnp.float32),
            jax.ShapeDtypeStruct((EPAD, H), jnp.float32),
        ],
        mesh=mesh,
        scratch_types=buf + buf,
    )
    return f(sndp_t, rcvp_t, senders_p, receivers_g)


def _count_body(rs_hbm, zcnt_hbm, cnt_out, idx_c, ones_v, cnt_sh):
    cid = lax.axis_index("c")
    sid = lax.axis_index("s")
    wid = sid * NC + cid
    base = wid * PER_W
    rows = NSEG // NS

    @pl.loop(0, W)
    def _(i):
        @pl.loop(0, D, step=16)
        def _(j):
            ones_v[i, pl.ds(j, 16)] = jnp.full((16,), 1.0, jnp.float32)

    pltpu.sync_copy(zcnt_hbm.at[pl.ds(sid * rows, rows)],
                    cnt_sh.at[pl.ds(sid * rows, rows)])
    plsc.subcore_barrier()

    @pl.loop(0, PER_W, step=W)
    def _(i):
        pltpu.sync_copy(rs_hbm.at[pl.ds(base + i, W)], idx_c)
        pltpu.sync_copy(ones_v, cnt_sh.at[idx_c], add=True)

    plsc.subcore_barrier()
    pltpu.sync_copy(cnt_sh.at[pl.ds(sid * rows, rows)],
                    cnt_out.at[cid].at[pl.ds(sid * rows, rows)])


def _sc_count(receivers_s, zcnt):
    mesh = plsc.VectorSubcoreMesh(core_axis_name="c", subcore_axis_name="s")
    f = pl.kernel(
        _count_body,
        out_type=jax.ShapeDtypeStruct((NC, NSEG, D), jnp.float32),
        mesh=mesh,
        scratch_types=[
            pltpu.VMEM((W,), jnp.int32),
            pltpu.VMEM((W, D), jnp.float32),
            pltpu.VMEM_SHARED((NSEG, D), jnp.float32),
        ],
    )
    return f(receivers_s, zcnt)


def _edge_body(ef_ref, sp_ref, rp_ref, we1_ref, be1_ref, we2_ref, be2_ref,
               wu1a_ref, bu1_ref, wu2_ref, bu2_ref, out_ref):
    x = ef_ref[...]
    h1 = jnp.dot(x, we1_ref[...], preferred_element_type=jnp.float32)
    h1 = _swish(h1 + be1_ref[...])
    e0 = _bdot(h1, we2_ref[...])
    e0 = _ln(e0 + be2_ref[...])
    pre = _bdot(e0, wu1a_ref[...])
    pre = _swish(pre + sp_ref[...] + rp_ref[...] + bu1_ref[...])
    u = _bdot(pre, wu2_ref[...])
    out_ref[...] = e0 + _ln(u + bu2_ref[...])


def _edge_mlp(ef_p, sndp, rcvp, we1, be1, we2, be2, wu1a, bu1, wu2, bu2):
    tile = 1024
    grid = EPAD // tile
    full = lambda i: (0, 0)
    return pl.pallas_call(
        _edge_body,
        grid=(grid,),
        in_specs=[
            pl.BlockSpec((tile, DE), lambda i: (i, 0)),
            pl.BlockSpec((tile, H), lambda i: (i, 0)),
            pl.BlockSpec((tile, H), lambda i: (i, 0)),
            pl.BlockSpec((DE, H), full),
            pl.BlockSpec((1, H), full),
            pl.BlockSpec((H, D), full),
            pl.BlockSpec((1, D), full),
            pl.BlockSpec((D, H), full),
            pl.BlockSpec((1, H), full),
            pl.BlockSpec((H, D), full),
            pl.BlockSpec((1, D), full),
        ],
        out_specs=pl.BlockSpec((tile, D), lambda i: (i, 0)),
        out_shape=jax.ShapeDtypeStruct((EPAD, D), jnp.float32),
        compiler_params=pltpu.CompilerParams(
            dimension_semantics=("parallel",)),
    )(ef_p, sndp, rcvp, we1, be1, we2, be2, wu1a, bu1, wu2, bu2)


def _scatter_body(vals_hbm, ridx_hbm, zseg_hbm, seg_out,
                  idx_v0, vals_v0, sem0, idx_v1, vals_v1, sem1, seg_sh):
    cid = lax.axis_index("c")
    sid = lax.axis_index("s")
    wid = sid * NC + cid
    base = wid * PER_W
    rows = NSEG // NS

    pltpu.sync_copy(zseg_hbm.at[pl.ds(sid * rows, rows)],
                    seg_sh.at[pl.ds(sid * rows, rows)])
    plsc.subcore_barrier()

    def start(off, idx_v, vals_v, sem):
        pltpu.sync_copy(ridx_hbm.at[pl.ds(off, W)], idx_v)
        pltpu.async_copy(vals_hbm.at[pl.ds(off, W)], vals_v, sem)

    def drain(idx_v, vals_v, sem):
        pltpu.make_async_copy(vals_hbm.at[pl.ds(0, W)], vals_v, sem).wait()
        pltpu.sync_copy(vals_v, seg_sh.at[idx_v], add=True)

    b0 = (idx_v0, vals_v0, sem0)
    b1 = (idx_v1, vals_v1, sem1)

    start(base, *b0)

    @pl.loop(0, PER_W - 2 * W, step=2 * W)
    def _(i):
        start(base + i + W, *b1)
        drain(*b0)
        start(base + i + 2 * W, *b0)
        drain(*b1)

    start(base + PER_W - W, *b1)
    drain(*b0)
    drain(*b1)

    plsc.subcore_barrier()

    pltpu.sync_copy(seg_sh.at[pl.ds(sid * rows, rows)],
                    seg_out.at[cid].at[pl.ds(sid * rows, rows)])


def _sc_scatter(vals, receivers_s, zseg):
    mesh = plsc.VectorSubcoreMesh(core_axis_name="c", subcore_axis_name="s")
    buf = [
        pltpu.VMEM((W,), jnp.int32),
        pltpu.VMEM((W, D), jnp.float32),
        pltpu.SemaphoreType.DMA,
    ]
    f = pl.kernel(
        _scatter_body,
        out_type=jax.ShapeDtypeStruct((NC, NSEG, D), jnp.float32),
        mesh=mesh,
        scratch_types=buf + buf + [pltpu.VMEM_SHARED((NSEG, D), jnp.float32)],
    )
    return f(vals, receivers_s, zseg)


def _node_body(pn_ref, seg_ref, cnt_ref, wn1a_ref, wn1b_ref, bn1_ref,
               wn2_ref, bn2_ref, wo1_ref, bo1_ref, wo2_ref, bo2_ref,
               out_ref):
    pn = pn_ref[...]
    seg = seg_ref[0] + seg_ref[1]
    cnt = cnt_ref[0] + cnt_ref[1]
    agg = seg / jnp.maximum(cnt, 1.0)
    pre = _bdot(pn, wn1a_ref[...]) + _bdot(agg, wn1b_ref[...])
    h = _swish(pre + bn1_ref[...])
    u = _bdot(h, wn2_ref[...])
    node = pn + _ln(u + bn2_ref[...])
    h2 = _swish(_bdot(node, wo1_ref[...]) + bo1_ref[...])
    out_ref[...] = _bdot(h2, wo2_ref[...]) + bo2_ref[...]


def _node_mlp(pn, seg_parts, cnt_parts, wn1a, wn1b, bn1, wn2, bn2,
              wo1, bo1, wo2, bo2):
    tile = 2000
    grid = N // tile
    full = lambda i: (0, 0)
    return pl.pallas_call(
        _node_body,
        grid=(grid,),
        in_specs=[
            pl.BlockSpec((tile, D), lambda i: (i, 0)),
            pl.BlockSpec((NC, tile, D), lambda i: (0, i, 0)),
            pl.BlockSpec((NC, tile, D), lambda i: (0, i, 0)),
            pl.BlockSpec((D, H), full),
            pl.BlockSpec((D, H), full),
            pl.BlockSpec((1, H), full),
            pl.BlockSpec((H, D), full),
            pl.BlockSpec((1, D), full),
            pl.BlockSpec((D, H), full),
            pl.BlockSpec((1, H), full),
            pl.BlockSpec((H, OUT), full),
            pl.BlockSpec((1, OUT), full),
        ],
        out_specs=pl.BlockSpec((tile, OUT), lambda i: (i, 0)),
        out_shape=jax.ShapeDtypeStruct((N, OUT), jnp.float32),
        compiler_params=pltpu.CompilerParams(
            dimension_semantics=("parallel",)),
    )(pn, seg_parts, cnt_parts, wn1a, wn1b, bn1, wn2, bn2,
      wo1, bo1, wo2, bo2)


def kernel(rnode_features, pnode_features, edge_features, senders, receivers,
           We1, be1, We2, be2, Wu1, bu1, Wu2, bu2,
           Wn1, bn1, Wn2, bn2, Wo1, bo1, Wo2, bo2):
    rn = rnode_features.reshape(N, D)
    pn = pnode_features.reshape(N, D)
    ef = edge_features.astype(jnp.float32)

    pad_g = (jnp.arange(PAD, dtype=jnp.int32) * 37) % N
    pad_s = N + (jnp.arange(PAD, dtype=jnp.int32) % DUMMY)
    senders_p = jnp.concatenate([senders, pad_g])
    receivers_g = jnp.concatenate([receivers, pad_g])
    receivers_s = jnp.concatenate([receivers, pad_s])
    ef_p = jnp.zeros((EPAD, DE), jnp.float32).at[:E].set(ef)

    wu1a, wu1b, wu1c = Wu1[:D], Wu1[D:2 * D], Wu1[2 * D:]
    wn1a, wn1b = Wn1[:D], Wn1[D:]
    r2 = lambda b: b.reshape(1, -1)

    zseg = jnp.zeros((NSEG, D), jnp.float32)
    sndp_t, rcvp_t = _node_proj(rn, pn, wu1b, wu1c)
    sndp, rcvp = _sc_gather(sndp_t, rcvp_t, senders_p, receivers_g)
    cnt_parts = _sc_count(receivers_s, zseg)
    e1 = _edge_mlp(ef_p, sndp, rcvp, We1, r2(be1), We2, r2(be2),
                   wu1a, r2(bu1), Wu2, r2(bu2))
    seg_parts = _sc_scatter(e1, receivers_s, zseg)
    out = _node_mlp(pn, seg_parts, cnt_parts, wn1a, wn1b, r2(bn1),
                    Wn2, r2(bn2), Wo1, r2(bo1), Wo2, r2(bo2))
    return out.reshape(N, 1, OUT)
